# Initial kernel scaffold; baseline (speedup 1.0000x reference)
import jax
import jax.numpy as jnp
from jax.experimental import pallas as pl


def kernel(x, edge_index, W, b):
    raise NotImplementedError("write your pallas kernel here")


# speedup vs baseline: 17.8033x; 17.8033x over previous
import functools

import jax
import jax.numpy as jnp
from jax import lax
from jax.experimental import pallas as pl
from jax.experimental.pallas import tpu as pltpu
from jax.experimental.pallas import tpu_sc as plsc

_EPS = 1e-6
_NC = 2
_NS = 16
_NW = _NC * _NS
_K = 80


def _sc_degree(col, n_pad, e):
    ept = e // _NW
    n_iter = ept // _K
    rpt = n_pad // _NS

    mesh = plsc.VectorSubcoreMesh(core_axis_name="c", subcore_axis_name="s")

    @functools.partial(
        pl.kernel,
        out_type=jax.ShapeDtypeStruct((_NC * n_pad,), jnp.float32),
        mesh=mesh,
        scratch_types=[
            pltpu.VMEM((_K,), jnp.int32),
            pltpu.VMEM((_K,), jnp.float32),
            pltpu.VMEM((rpt,), jnp.float32),
            pltpu.VMEM_SHARED((n_pad,), jnp.float32),
        ],
    )
    def deg_kernel(col_hbm, out_hbm, idx_v, ones_v, zb_v, hist_sh):
        c = lax.axis_index("c")
        s = lax.axis_index("s")
        z16 = jnp.zeros((16,), jnp.float32)
        o16 = jnp.ones((16,), jnp.float32)
        for j in range(_K // 16):
            ones_v[pl.ds(j * 16, 16)] = o16
        for j in range(rpt // 16):
            zb_v[pl.ds(j * 16, 16)] = z16
        pltpu.sync_copy(zb_v, hist_sh.at[pl.ds(s * rpt, rpt)])
        plsc.subcore_barrier()
        e0 = (c * _NS + s) * ept

        def body(i, carry):
            pltpu.sync_copy(col_hbm.at[pl.ds(e0 + i * _K, _K)], idx_v)
            pltpu.sync_copy(ones_v, hist_sh.at[idx_v], add=True)
            return carry

        lax.fori_loop(0, n_iter, body, 0)
        plsc.subcore_barrier()
        pltpu.sync_copy(hist_sh.at[pl.ds(s * rpt, rpt)], zb_v)
        pltpu.sync_copy(zb_v, out_hbm.at[pl.ds(c * n_pad + s * rpt, rpt)])

    return deg_kernel(col)


def _sc_propagate(g, row, col, n_pad, e):
    d = g.shape[1]
    ept = e // _NW
    n_iter = ept // _K
    rpt = n_pad // _NS
    zr = 16

    mesh = plsc.VectorSubcoreMesh(core_axis_name="c", subcore_axis_name="s")

    @functools.partial(
        pl.kernel,
        out_type=jax.ShapeDtypeStruct((_NC * n_pad, d), jnp.float32),
        mesh=mesh,
        scratch_types=[
            pltpu.VMEM((_K,), jnp.int32),
            pltpu.VMEM((_K,), jnp.int32),
            pltpu.VMEM((_K, d), jnp.float32),
            pltpu.VMEM((zr, d), jnp.float32),
            pltpu.VMEM_SHARED((n_pad, d), jnp.float32),
        ],
    )
    def prop_kernel(g_hbm, row_hbm, col_hbm, out_hbm,
                    rid_v, cid_v, rows_v, zb_v, acc_sh):
        c = lax.axis_index("c")
        s = lax.axis_index("s")
        z16 = jnp.zeros((16,), jnp.float32)
        for r in range(zr):
            for j in range(d // 16):
                zb_v[r, pl.ds(j * 16, 16)] = z16
        r0 = s * rpt

        def zero_body(i, carry):
            pltpu.sync_copy(zb_v, acc_sh.at[pl.ds(r0 + i * zr, zr)])
            return carry

        lax.fori_loop(0, rpt // zr, zero_body, 0)
        plsc.subcore_barrier()
        e0 = (c * _NS + s) * ept

        def body(i, carry):
            pltpu.sync_copy(row_hbm.at[pl.ds(e0 + i * _K, _K)], rid_v)
            pltpu.sync_copy(col_hbm.at[pl.ds(e0 + i * _K, _K)], cid_v)
            pltpu.sync_copy(g_hbm.at[rid_v], rows_v)
            pltpu.sync_copy(rows_v, acc_sh.at[cid_v], add=True)
            return carry

        lax.fori_loop(0, n_iter, body, 0)
        plsc.subcore_barrier()

        def wb_body(i, carry):
            pltpu.sync_copy(acc_sh.at[pl.ds(r0 + i * _K, _K)], rows_v)
            pltpu.sync_copy(rows_v, out_hbm.at[pl.ds(c * n_pad + r0 + i * _K, _K)])
            return carry

        lax.fori_loop(0, rpt // _K, wb_body, 0)

    return prop_kernel(g, row, col)


def _tc_transform(x, W, deg2):
    n, d = x.shape
    bn = 1000

    def body(x_ref, w_ref, deg_ref, g_ref, dinv_ref):
        dg = deg_ref[...]
        deg = dg[:, 0:1] + dg[:, 1:2] + 1.0
        dinv = lax.rsqrt(deg)
        h = jnp.dot(x_ref[...], w_ref[...], preferred_element_type=jnp.float32)
        g_ref[...] = h * dinv
        dinv_ref[...] = dinv

    return pl.pallas_call(
        body,
        grid=(n // bn,),
        in_specs=[pl.BlockSpec((bn, d), lambda i: (i, 0)),
                  pl.BlockSpec((d, d), lambda i: (0, 0)),
                  pl.BlockSpec((bn, 2), lambda i: (i, 0))],
        out_specs=[pl.BlockSpec((bn, d), lambda i: (i, 0)),
                   pl.BlockSpec((bn, 1), lambda i: (i, 0))],
        out_shape=[jax.ShapeDtypeStruct((n, d), jnp.float32),
                   jax.ShapeDtypeStruct((n, 1), jnp.float32)],
    )(x, W, deg2)


def _tc_finish(p, g, dinv, b, n):
    d = g.shape[1]
    bn = 1000

    def body(p_ref, g_ref, dinv_ref, b_ref, o_ref):
        sacc = p_ref[0] + p_ref[1] + g_ref[...]
        o = sacc * dinv_ref[...] + b_ref[...]
        mu = jnp.mean(o, axis=1, keepdims=True)
        var = jnp.mean((o - mu) ** 2, axis=1, keepdims=True)
        o = (o - mu) * lax.rsqrt(var + _EPS)
        o_ref[...] = jnp.where(o >= 0, o, 0.01 * o)

    return pl.pallas_call(
        body,
        grid=(n // bn,),
        in_specs=[pl.BlockSpec((2, bn, d), lambda i: (0, i, 0)),
                  pl.BlockSpec((bn, d), lambda i: (i, 0)),
                  pl.BlockSpec((bn, 1), lambda i: (i, 0)),
                  pl.BlockSpec((1, d), lambda i: (0, 0))],
        out_specs=pl.BlockSpec((bn, d), lambda i: (i, 0)),
        out_shape=jax.ShapeDtypeStruct((n, d), jnp.float32),
    )(p, g, dinv, b.reshape(1, d))


def kernel(x, edge_index, W, b):
    n, d = x.shape
    e = edge_index.shape[1]
    n_pad = ((n + _NS * _K - 1) // (_NS * _K)) * (_NS * _K)
    row = edge_index[0]
    col = edge_index[1]
    degp = _sc_degree(col, n_pad, e).reshape(_NC, n_pad)
    deg2 = jnp.transpose(degp)[:n]
    g, dinv = _tc_transform(x, W, deg2)
    p = _sc_propagate(g, row, col, n_pad, e)
    return _tc_finish(p.reshape(_NC, n_pad, d), g, dinv, b, n)
